# baseline (speedup 1.0000x reference)
import functools

import jax
import jax.numpy as jnp
from jax import lax
from jax.experimental import pallas as pl
from jax.experimental.pallas import tpu as pltpu
from jax.experimental.pallas import tpu_sc as plsc

_IW = 128
_JROWS = 4
_CHUNK = _IW * _JROWS


def _make_out_transpose(b: int, s: int, d: int):
    nbg = b // _IW

    def body(in_ref, out_ref):
        blk = in_ref[...]
        row = lax.broadcasted_iota(jnp.int32, (d, _IW), 0)
        col = lax.broadcasted_iota(jnp.int32, (d, _IW), 1)
        s_even = (col == 2 * row).astype(jnp.float32)
        s_odd = (col == 2 * row + 1).astype(jnp.float32)
        dn = (((0,), (0,)), ((), ()))
        out_ref[0] = (
            lax.dot_general(blk[:, :d], s_even, dn,
                            preferred_element_type=jnp.float32)
            + lax.dot_general(blk[:, d:], s_odd, dn,
                              preferred_element_type=jnp.float32))

    return pl.pallas_call(
        body,
        grid=(s, nbg),
        in_specs=[pl.BlockSpec((d, 2 * d), lambda i, j: (i * nbg + j, 0))],
        out_specs=pl.BlockSpec((1, d, _IW), lambda i, j: (i, 0, j)),
        out_shape=jax.ShapeDtypeStruct((s, d, b), jnp.float32),
    )


def _make_gather(n_rows: int, n_vocab: int, d: int):
    info = plsc.get_sparse_core_info()
    nw = info.num_cores * info.num_subcores
    nc = info.num_cores
    rows_per_w = n_rows // nw
    idx_rows_per_w = rows_per_w // _IW
    chunks = idx_rows_per_w // _JROWS
    assert rows_per_w * nw == n_rows and chunks * _JROWS == idx_rows_per_w
    assert chunks % 2 == 0 and chunks >= 4

    mesh = plsc.VectorSubcoreMesh(core_axis_name="c", subcore_axis_name="s")

    @functools.partial(
        pl.kernel,
        mesh=mesh,
        compiler_params=pltpu.CompilerParams(use_tc_tiling_on_sc=False),
        out_type=jax.ShapeDtypeStruct((n_rows, d), jnp.float32),
        scratch_types=[
            pltpu.VMEM((idx_rows_per_w, _IW), jnp.int32),
            pltpu.VMEM((_CHUNK, d), jnp.float32),
            pltpu.VMEM((_CHUNK, d), jnp.float32),
            pltpu.SemaphoreType.DMA,
            pltpu.SemaphoreType.DMA,
        ],
    )
    def k(table_hbm, idx_hbm, out_hbm, idx_v, rows0, rows1, sem0, sem1):
        wid = lax.axis_index("s") * nc + lax.axis_index("c")
        idx_row0 = wid * idx_rows_per_w
        out_row0 = wid * rows_per_w

        pltpu.sync_copy(idx_hbm.at[pl.ds(idx_row0, idx_rows_per_w)], idx_v)

        def fire(c, rows_v, sem):
            for j in range(_JROWS):
                pltpu.async_copy(table_hbm.at[idx_v.at[c * _JROWS + j]],
                                 rows_v.at[pl.ds(j * _IW, _IW)], sem)

        def drain_and_write(c, rows_v, sem):
            for j in range(_JROWS):
                pltpu.make_async_copy(
                    table_hbm.at[idx_v.at[j]],
                    rows_v.at[pl.ds(j * _IW, _IW)], sem).wait()
            pltpu.sync_copy(rows_v,
                            out_hbm.at[pl.ds(out_row0 + c * _CHUNK, _CHUNK)])

        fire(0, rows0, sem0)

        def body(i, carry):
            c = 2 * i
            fire(c + 1, rows1, sem1)
            drain_and_write(c, rows0, sem0)
            fire(c + 2, rows0, sem0)
            drain_and_write(c + 1, rows1, sem1)
            return carry

        lax.fori_loop(0, chunks // 2 - 1, body, 0)

        c = chunks - 2
        fire(c + 1, rows1, sem1)
        drain_and_write(c, rows0, sem0)
        drain_and_write(c + 1, rows1, sem1)

    return k


def kernel(x, weight):
    b, s = x.shape
    n = b * s
    v, d = weight.shape
    w2 = jnp.pad(weight, ((0, 0), (0, 128 - d))).reshape(2 * v, d)
    idx2d = (x.T.reshape(n // _IW, _IW) * 2).astype(jnp.int32)
    out_lin = _make_gather(n, 2 * v, d)(w2, idx2d)
    out_nat = _make_out_transpose(b, s, d)(out_lin.reshape(n // 2, 2 * d))
    return out_nat.transpose(2, 0, 1)


# speedup vs baseline: 4.2492x; 4.2492x over previous
import functools

import jax
import jax.numpy as jnp
from jax import lax
from jax.experimental import pallas as pl
from jax.experimental.pallas import tpu as pltpu
from jax.experimental.pallas import tpu_sc as plsc

_IW = 128
_JROWS = 4
_CHUNK = _IW * _JROWS


def _make_gather(n_rows: int, n_vocab: int, d: int):
    info = plsc.get_sparse_core_info()
    nw = info.num_cores * info.num_subcores
    nc = info.num_cores
    rows_per_w = n_rows // nw
    idx_rows_per_w = rows_per_w // _IW
    chunks = idx_rows_per_w // _JROWS
    assert rows_per_w * nw == n_rows and chunks * _JROWS == idx_rows_per_w
    assert chunks % 2 == 0 and chunks >= 4

    mesh = plsc.VectorSubcoreMesh(core_axis_name="c", subcore_axis_name="s")

    @functools.partial(
        pl.kernel,
        mesh=mesh,
        compiler_params=pltpu.CompilerParams(use_tc_tiling_on_sc=False),
        out_type=jax.ShapeDtypeStruct((n_rows, d), jnp.float32),
        scratch_types=[
            pltpu.VMEM((idx_rows_per_w, _IW), jnp.int32),
            pltpu.VMEM((_CHUNK, d), jnp.float32),
            pltpu.VMEM((_CHUNK, d), jnp.float32),
            pltpu.SemaphoreType.DMA,
            pltpu.SemaphoreType.DMA,
        ],
    )
    def k(table_hbm, idx_hbm, out_hbm, idx_v, rows0, rows1, sem0, sem1):
        wid = lax.axis_index("s") * nc + lax.axis_index("c")
        idx_row0 = wid * idx_rows_per_w
        out_row0 = wid * rows_per_w

        pltpu.sync_copy(idx_hbm.at[pl.ds(idx_row0, idx_rows_per_w)], idx_v)

        def fire(c, rows_v, sem):
            for j in range(_JROWS):
                pltpu.async_copy(table_hbm.at[idx_v.at[c * _JROWS + j]],
                                 rows_v.at[pl.ds(j * _IW, _IW)], sem)

        def drain_and_write(c, rows_v, sem):
            for j in range(_JROWS):
                pltpu.make_async_copy(
                    table_hbm.at[idx_v.at[j]],
                    rows_v.at[pl.ds(j * _IW, _IW)], sem).wait()
            pltpu.sync_copy(rows_v,
                            out_hbm.at[pl.ds(out_row0 + c * _CHUNK, _CHUNK)])

        fire(0, rows0, sem0)

        def body(i, carry):
            c = 2 * i
            fire(c + 1, rows1, sem1)
            drain_and_write(c, rows0, sem0)
            fire(c + 2, rows0, sem0)
            drain_and_write(c + 1, rows1, sem1)
            return carry

        lax.fori_loop(0, chunks // 2 - 1, body, 0)

        c = chunks - 2
        fire(c + 1, rows1, sem1)
        drain_and_write(c, rows0, sem0)
        drain_and_write(c + 1, rows1, sem1)

    return k


def kernel(x, weight):
    b, s = x.shape
    n = b * s
    v, d = weight.shape
    w2 = jnp.pad(weight, ((0, 0), (0, 128 - d))).reshape(2 * v, d)
    idx2d = (x.reshape(n // _IW, _IW) * 2).astype(jnp.int32)
    out_lin = _make_gather(n, 2 * v, d)(w2, idx2d)
    out_t = out_lin.reshape(b, s * d).T
    return out_t.reshape(s, d, b).transpose(2, 0, 1)


# speedup vs baseline: 4.3832x; 1.0315x over previous
import functools

import jax
import jax.numpy as jnp
from jax import lax
from jax.experimental import pallas as pl
from jax.experimental.pallas import tpu as pltpu
from jax.experimental.pallas import tpu_sc as plsc

_IW = 128
_JROWS = 4
_CHUNK = _IW * _JROWS


def _make_table_prep(v: int, d: int):
    bc = 4096
    grid = (v + bc - 1) // bc

    def body(in_ref, out_ref):
        blk = in_ref[...]
        eye = jnp.eye(d, dtype=jnp.float32)
        tr = lax.dot_general(blk, eye, (((0,), (0,)), ((), ())),
                             precision=lax.Precision.HIGHEST,
                             preferred_element_type=jnp.float32)
        out_ref[:, :d] = tr

    return pl.pallas_call(
        body,
        grid=(grid,),
        in_specs=[pl.BlockSpec((d, bc), lambda j: (0, j))],
        out_specs=pl.BlockSpec((bc, 128), lambda j: (j, 0)),
        out_shape=jax.ShapeDtypeStruct((v, 128), jnp.float32),
    )


def _make_gather(n_rows: int, n_vocab: int, d: int):
    info = plsc.get_sparse_core_info()
    nw = info.num_cores * info.num_subcores
    nc = info.num_cores
    rows_per_w = n_rows // nw
    idx_rows_per_w = rows_per_w // _IW
    chunks = idx_rows_per_w // _JROWS
    assert rows_per_w * nw == n_rows and chunks * _JROWS == idx_rows_per_w
    assert chunks % 2 == 0 and chunks >= 4

    mesh = plsc.VectorSubcoreMesh(core_axis_name="c", subcore_axis_name="s")

    @functools.partial(
        pl.kernel,
        mesh=mesh,
        compiler_params=pltpu.CompilerParams(use_tc_tiling_on_sc=False),
        out_type=jax.ShapeDtypeStruct((n_rows, d), jnp.float32),
        scratch_types=[
            pltpu.VMEM((idx_rows_per_w, _IW), jnp.int32),
            pltpu.VMEM((_CHUNK, d), jnp.float32),
            pltpu.VMEM((_CHUNK, d), jnp.float32),
            pltpu.SemaphoreType.DMA,
            pltpu.SemaphoreType.DMA,
        ],
    )
    def k(table_hbm, idx_hbm, out_hbm, idx_v, rows0, rows1, sem0, sem1):
        wid = lax.axis_index("s") * nc + lax.axis_index("c")
        idx_row0 = wid * idx_rows_per_w
        out_row0 = wid * rows_per_w

        pltpu.sync_copy(idx_hbm.at[pl.ds(idx_row0, idx_rows_per_w)], idx_v)

        def fire(c, rows_v, sem):
            for j in range(_JROWS):
                pltpu.async_copy(table_hbm.at[idx_v.at[c * _JROWS + j]],
                                 rows_v.at[pl.ds(j * _IW, _IW)], sem)

        def drain_and_write(c, rows_v, sem):
            for j in range(_JROWS):
                pltpu.make_async_copy(
                    table_hbm.at[idx_v.at[j]],
                    rows_v.at[pl.ds(j * _IW, _IW)], sem).wait()
            pltpu.sync_copy(rows_v,
                            out_hbm.at[pl.ds(out_row0 + c * _CHUNK, _CHUNK)])

        fire(0, rows0, sem0)

        def body(i, carry):
            c = 2 * i
            fire(c + 1, rows1, sem1)
            drain_and_write(c, rows0, sem0)
            fire(c + 2, rows0, sem0)
            drain_and_write(c + 1, rows1, sem1)
            return carry

        lax.fori_loop(0, chunks // 2 - 1, body, 0)

        c = chunks - 2
        fire(c + 1, rows1, sem1)
        drain_and_write(c, rows0, sem0)
        drain_and_write(c + 1, rows1, sem1)

    return k


def kernel(x, weight):
    b, s = x.shape
    n = b * s
    v, d = weight.shape
    w2 = _make_table_prep(v, d)(weight.T).reshape(2 * v, d)
    idx2d = (x.reshape(n // _IW, _IW) * 2).astype(jnp.int32)
    out_lin = _make_gather(n, 2 * v, d)(w2, idx2d)
    out_t = out_lin.reshape(b, s * d).T
    return out_t.reshape(s, d, b).transpose(2, 0, 1)


# speedup vs baseline: 4.5871x; 1.0465x over previous
import functools

import jax
import jax.numpy as jnp
from jax import lax
from jax.experimental import pallas as pl
from jax.experimental.pallas import tpu as pltpu
from jax.experimental.pallas import tpu_sc as plsc

_IW = 128
_JROWS = 4
_CHUNK = _IW * _JROWS


def _make_table_prep(v: int, d: int):
    bc = 8192
    grid = (v + bc - 1) // bc

    def body(in_ref, out_ref):
        blk = in_ref[...]
        eye = jnp.eye(d, dtype=jnp.float32)
        tr = lax.dot_general(blk, eye, (((0,), (0,)), ((), ())),
                             precision=lax.Precision.HIGHEST,
                             preferred_element_type=jnp.float32)
        out_ref[:, :d] = tr

    return pl.pallas_call(
        body,
        grid=(grid,),
        in_specs=[pl.BlockSpec((d, bc), lambda j: (0, j))],
        out_specs=pl.BlockSpec((bc, 2 * d), lambda j: (j, 0)),
        out_shape=jax.ShapeDtypeStruct((v, 2 * d), jnp.float32),
    )


def _make_gather(n_rows: int, n_vocab: int, d: int):
    info = plsc.get_sparse_core_info()
    nw = info.num_cores * info.num_subcores
    nc = info.num_cores
    rows_per_w = n_rows // nw
    idx_rows_per_w = rows_per_w // _IW
    chunks = idx_rows_per_w // _JROWS
    assert rows_per_w * nw == n_rows and chunks * _JROWS == idx_rows_per_w
    assert chunks % 2 == 0 and chunks >= 4

    mesh = plsc.VectorSubcoreMesh(core_axis_name="c", subcore_axis_name="s")

    @functools.partial(
        pl.kernel,
        mesh=mesh,
        compiler_params=pltpu.CompilerParams(use_tc_tiling_on_sc=False),
        out_type=jax.ShapeDtypeStruct((n_rows, d), jnp.float32),
        scratch_types=[
            pltpu.VMEM((idx_rows_per_w, _IW), jnp.int32),
            pltpu.VMEM((_CHUNK, d), jnp.float32),
            pltpu.VMEM((_CHUNK, d), jnp.float32),
            pltpu.SemaphoreType.DMA,
            pltpu.SemaphoreType.DMA,
        ],
    )
    def k(table_hbm, idx_hbm, out_hbm, idx_v, rows0, rows1, sem0, sem1):
        wid = lax.axis_index("s") * nc + lax.axis_index("c")
        idx_row0 = wid * idx_rows_per_w
        out_row0 = wid * rows_per_w

        pltpu.sync_copy(idx_hbm.at[pl.ds(idx_row0, idx_rows_per_w)], idx_v)

        def fire(c, rows_v, sem):
            for j in range(_JROWS):
                pltpu.async_copy(table_hbm.at[idx_v.at[c * _JROWS + j]],
                                 rows_v.at[pl.ds(j * _IW, _IW)], sem)

        def drain_and_write(c, rows_v, sem):
            for j in range(_JROWS):
                pltpu.make_async_copy(
                    table_hbm.at[idx_v.at[j]],
                    rows_v.at[pl.ds(j * _IW, _IW)], sem).wait()
            pltpu.sync_copy(rows_v,
                            out_hbm.at[pl.ds(out_row0 + c * _CHUNK, _CHUNK)])

        fire(0, rows0, sem0)

        def body(i, carry):
            c = 2 * i
            fire(c + 1, rows1, sem1)
            drain_and_write(c, rows0, sem0)
            fire(c + 2, rows0, sem0)
            drain_and_write(c + 1, rows1, sem1)
            return carry

        lax.fori_loop(0, chunks // 2 - 1, body, 0)

        c = chunks - 2
        fire(c + 1, rows1, sem1)
        drain_and_write(c, rows0, sem0)
        drain_and_write(c + 1, rows1, sem1)

    return k


def kernel(x, weight):
    b, s = x.shape
    n = b * s
    v, d = weight.shape
    w2 = _make_table_prep(v, d)(weight.T).reshape(2 * v, d)
    idx2d = (x.reshape(n // _IW, _IW) * 2).astype(jnp.int32)
    out_lin = _make_gather(n, 2 * v, d)(w2, idx2d)
    out_t = out_lin.reshape(b, s * d).T
    return out_t.reshape(s, d, b).transpose(2, 0, 1)


# speedup vs baseline: 5.5831x; 1.2171x over previous
import functools

import jax
import jax.numpy as jnp
from jax import lax
from jax.experimental import pallas as pl
from jax.experimental.pallas import tpu as pltpu
from jax.experimental.pallas import tpu_sc as plsc

_IW = 128
_JROWS = 4
_CHUNK = _IW * _JROWS


def _make_table_prep(v: int, d: int):
    bc = 8192
    grid = (v + bc - 1) // bc

    def body(in_ref, out_ref):
        out_ref[:, :d] = in_ref[...].T

    return pl.pallas_call(
        body,
        grid=(grid,),
        in_specs=[pl.BlockSpec((d, bc), lambda j: (0, j))],
        out_specs=pl.BlockSpec((bc, 2 * d), lambda j: (j, 0)),
        out_shape=jax.ShapeDtypeStruct((v, 2 * d), jnp.float32),
    )


def _make_gather(n_rows: int, n_vocab: int, d: int):
    info = plsc.get_sparse_core_info()
    nw = info.num_cores * info.num_subcores
    nc = info.num_cores
    rows_per_w = n_rows // nw
    idx_rows_per_w = rows_per_w // _IW
    chunks = idx_rows_per_w // _JROWS
    assert rows_per_w * nw == n_rows and chunks * _JROWS == idx_rows_per_w
    assert chunks % 2 == 0 and chunks >= 4

    mesh = plsc.VectorSubcoreMesh(core_axis_name="c", subcore_axis_name="s")

    @functools.partial(
        pl.kernel,
        mesh=mesh,
        compiler_params=pltpu.CompilerParams(use_tc_tiling_on_sc=False),
        out_type=jax.ShapeDtypeStruct((n_rows, d), jnp.float32),
        scratch_types=[
            pltpu.VMEM((idx_rows_per_w, _IW), jnp.int32),
            pltpu.VMEM((_CHUNK, d), jnp.float32),
            pltpu.VMEM((_CHUNK, d), jnp.float32),
            pltpu.SemaphoreType.DMA,
            pltpu.SemaphoreType.DMA,
        ],
    )
    def k(table_hbm, idx_hbm, out_hbm, idx_v, rows0, rows1, sem0, sem1):
        wid = lax.axis_index("s") * nc + lax.axis_index("c")
        idx_row0 = wid * idx_rows_per_w
        out_row0 = wid * rows_per_w

        pltpu.sync_copy(idx_hbm.at[pl.ds(idx_row0, idx_rows_per_w)], idx_v)

        def fire(c, rows_v, sem):
            for j in range(_JROWS):
                pltpu.async_copy(table_hbm.at[idx_v.at[c * _JROWS + j]],
                                 rows_v.at[pl.ds(j * _IW, _IW)], sem)

        def drain_and_write(c, rows_v, sem):
            for j in range(_JROWS):
                pltpu.make_async_copy(
                    table_hbm.at[idx_v.at[j]],
                    rows_v.at[pl.ds(j * _IW, _IW)], sem).wait()
            pltpu.sync_copy(rows_v,
                            out_hbm.at[pl.ds(out_row0 + c * _CHUNK, _CHUNK)])

        fire(0, rows0, sem0)

        def body(i, carry):
            c = 2 * i
            fire(c + 1, rows1, sem1)
            drain_and_write(c, rows0, sem0)
            fire(c + 2, rows0, sem0)
            drain_and_write(c + 1, rows1, sem1)
            return carry

        lax.fori_loop(0, chunks // 2 - 1, body, 0)

        c = chunks - 2
        fire(c + 1, rows1, sem1)
        drain_and_write(c, rows0, sem0)
        drain_and_write(c + 1, rows1, sem1)

    return k


def kernel(x, weight):
    b, s = x.shape
    n = b * s
    v, d = weight.shape
    w2 = _make_table_prep(v, d)(weight.T).reshape(2 * v, d)
    idx2d = (x.reshape(n // _IW, _IW) * 2).astype(jnp.int32)
    out_lin = _make_gather(n, 2 * v, d)(w2, idx2d)
    out_t = out_lin.reshape(b, s * d).T
    return out_t.reshape(s, d, b).transpose(2, 0, 1)


# speedup vs baseline: 5.7321x; 1.0267x over previous
import functools

import jax
import jax.numpy as jnp
from jax import lax
from jax.experimental import pallas as pl
from jax.experimental.pallas import tpu as pltpu
from jax.experimental.pallas import tpu_sc as plsc

_IW = 128
_JROWS = 4
_CHUNK = _IW * _JROWS


def _make_table_prep(v: int, d: int):
    bc = 16384
    grid = (v + bc - 1) // bc

    def body(in_ref, out_ref):
        out_ref[:, :d] = in_ref[...].T

    return pl.pallas_call(
        body,
        grid=(grid,),
        in_specs=[pl.BlockSpec((d, bc), lambda j: (0, j))],
        out_specs=pl.BlockSpec((bc, 2 * d), lambda j: (j, 0)),
        out_shape=jax.ShapeDtypeStruct((v, 2 * d), jnp.float32),
    )


def _make_gather(n_rows: int, n_vocab: int, d: int):
    info = plsc.get_sparse_core_info()
    nw = info.num_cores * info.num_subcores
    nc = info.num_cores
    rows_per_w = n_rows // nw
    idx_rows_per_w = rows_per_w // _IW
    chunks = idx_rows_per_w // _JROWS
    assert rows_per_w * nw == n_rows and chunks * _JROWS == idx_rows_per_w
    assert chunks % 2 == 0 and chunks >= 4

    mesh = plsc.VectorSubcoreMesh(core_axis_name="c", subcore_axis_name="s")

    @functools.partial(
        pl.kernel,
        mesh=mesh,
        compiler_params=pltpu.CompilerParams(use_tc_tiling_on_sc=False),
        out_type=jax.ShapeDtypeStruct((n_rows, d), jnp.float32),
        scratch_types=[
            pltpu.VMEM((idx_rows_per_w, _IW), jnp.int32),
            pltpu.VMEM((_CHUNK, d), jnp.float32),
            pltpu.VMEM((_CHUNK, d), jnp.float32),
            pltpu.SemaphoreType.DMA,
            pltpu.SemaphoreType.DMA,
        ],
    )
    def k(table_hbm, idx_hbm, out_hbm, idx_v, rows0, rows1, sem0, sem1):
        wid = lax.axis_index("s") * nc + lax.axis_index("c")
        idx_row0 = wid * idx_rows_per_w
        out_row0 = wid * rows_per_w

        pltpu.sync_copy(idx_hbm.at[pl.ds(idx_row0, idx_rows_per_w)], idx_v)

        def fire(c, rows_v, sem):
            for j in range(_JROWS):
                pltpu.async_copy(table_hbm.at[idx_v.at[c * _JROWS + j]],
                                 rows_v.at[pl.ds(j * _IW, _IW)], sem)

        def drain_and_write(c, rows_v, sem):
            for j in range(_JROWS):
                pltpu.make_async_copy(
                    table_hbm.at[idx_v.at[j]],
                    rows_v.at[pl.ds(j * _IW, _IW)], sem).wait()
            pltpu.sync_copy(rows_v,
                            out_hbm.at[pl.ds(out_row0 + c * _CHUNK, _CHUNK)])

        fire(0, rows0, sem0)

        def body(i, carry):
            c = 2 * i
            fire(c + 1, rows1, sem1)
            drain_and_write(c, rows0, sem0)
            fire(c + 2, rows0, sem0)
            drain_and_write(c + 1, rows1, sem1)
            return carry

        lax.fori_loop(0, chunks // 2 - 1, body, 0)

        c = chunks - 2
        fire(c + 1, rows1, sem1)
        drain_and_write(c, rows0, sem0)
        drain_and_write(c + 1, rows1, sem1)

    return k


def kernel(x, weight):
    b, s = x.shape
    n = b * s
    v, d = weight.shape
    w2 = _make_table_prep(v, d)(weight.T).reshape(2 * v, d)
    idx2d = (x.reshape(n // _IW, _IW) * 2).astype(jnp.int32)
    out_lin = _make_gather(n, 2 * v, d)(w2, idx2d)
    out_t = out_lin.reshape(b, s * d).T
    return out_t.reshape(s, d, b).transpose(2, 0, 1)
